# baseline (speedup 1.0000x reference)
import jax
import jax.numpy as jnp
from jax import lax
from jax.experimental import pallas as pl
from jax.experimental.pallas import tpu as pltpu
from jax.experimental.pallas import tpu_sc as plsc

NR = 10000
NR_P = 10240
NSUB = 1000
NSUB_P = 1024
D = 128
H = 128
OUTD = 32
ERR = 320000
ERS = 160000

NC = 2
NS = 16
NW = NC * NS

CW = 128
NCH_RR = 80
NCH_RS = 40
G_RR = 16
G_RS = 8
NB = 5
ERR_P = NW * NCH_RR * CW
ERS_P = NW * NCH_RS * CW
ROWS_R = NR_P // NS
ROWS_S = NSUB_P // NS

_MESH = plsc.VectorSubcoreMesh(
    core_axis_name="c", subcore_axis_name="s", num_cores=NC, num_subcores=NS
)

f32 = jnp.float32


def _gs_loop(table, acc, src3, dst3, wid, sblk, dblk, rows0, rows1,
             sg0, sg1, ss0, ss1, nb, g):
    def blk(b, carry):
        pltpu.sync_copy(src3.at[wid, pl.ds(b * g, g)], sblk)
        pltpu.sync_copy(dst3.at[wid, pl.ds(b * g, g)], dblk)
        pending = [None, None]
        for j in range(g):
            buf, sg, ss = (rows0, sg0, ss0) if j % 2 == 0 else (rows1, sg1, ss1)
            if pending[j % 2] is not None:
                pending[j % 2].wait()
            pltpu.async_copy(table.at[sblk.at[j]], buf, sg).wait()
            pending[j % 2] = pltpu.async_copy(buf, acc.at[dblk.at[j]], ss,
                                              add=True)
        pending[0].wait()
        pending[1].wait()
        return carry

    lax.fori_loop(0, nb, blk, 0)


def _hist_loop(ones_v, acc, dst3, wid, dblk, shist, nb, g):
    def blk(b, carry):
        pltpu.sync_copy(dst3.at[wid, pl.ds(b * g, g)], dblk)
        pending = []
        for j in range(g):
            pending.append(
                pltpu.async_copy(ones_v, acc.at[dblk.at[j]], shist, add=True))
        for p in pending:
            p.wait()
        return carry

    lax.fori_loop(0, nb, blk, 0)


def _sc_pass1_body(dst_rr3, src_rs3, dst_rs3, x_region, z_deg, z_cnt, z_sum,
                   ones_hbm, deg_out, cnt_out, sum1_out,
                   deg_acc, cnt_acc, sum_acc,
                   dblk_rr, sblk, dblk, ones_v, rows0, rows1,
                   sg0, sg1, ss0, ss1, shist):
    c = lax.axis_index("c")
    s = lax.axis_index("s")
    wid = c * NS + s
    pltpu.sync_copy(ones_hbm, ones_v)
    pltpu.sync_copy(z_deg, deg_acc.at[pl.ds(s * ROWS_R, ROWS_R)])
    pltpu.sync_copy(z_cnt, cnt_acc.at[pl.ds(s * ROWS_S, ROWS_S)])
    pltpu.sync_copy(z_sum, sum_acc.at[pl.ds(s * ROWS_S, ROWS_S)])
    plsc.subcore_barrier()

    _hist_loop(ones_v, deg_acc, dst_rr3, wid, dblk_rr, shist, NB, G_RR)
    _hist_loop(ones_v, cnt_acc, dst_rs3, wid, dblk, shist, NB, G_RS)
    _gs_loop(x_region, sum_acc, src_rs3, dst_rs3, wid, sblk, dblk,
             rows0, rows1, sg0, sg1, ss0, ss1, NB, G_RS)

    plsc.subcore_barrier()
    pltpu.sync_copy(deg_acc.at[pl.ds(s * ROWS_R, ROWS_R)],
                    deg_out.at[c].at[pl.ds(s * ROWS_R, ROWS_R)])
    pltpu.sync_copy(cnt_acc.at[pl.ds(s * ROWS_S, ROWS_S)],
                    cnt_out.at[c].at[pl.ds(s * ROWS_S, ROWS_S)])
    pltpu.sync_copy(sum_acc.at[pl.ds(s * ROWS_S, ROWS_S)],
                    sum1_out.at[c].at[pl.ds(s * ROWS_S, ROWS_S)])


_sc_pass1 = pl.kernel(
    _sc_pass1_body,
    out_type=(
        jax.ShapeDtypeStruct((NC, NR_P, 16), f32),
        jax.ShapeDtypeStruct((NC, NSUB_P, 16), f32),
        jax.ShapeDtypeStruct((NC, NSUB_P, D), f32),
    ),
    mesh=_MESH,
    scratch_types=(
        pltpu.VMEM_SHARED((NR_P, 16), f32),
        pltpu.VMEM_SHARED((NSUB_P, 16), f32),
        pltpu.VMEM_SHARED((NSUB_P, D), f32),
        pltpu.VMEM((G_RR, CW), jnp.int32),
        pltpu.VMEM((G_RS, CW), jnp.int32),
        pltpu.VMEM((G_RS, CW), jnp.int32),
        pltpu.VMEM((CW, 16), f32),
        pltpu.VMEM((CW, D), f32),
        pltpu.VMEM((CW, D), f32),
        pltpu.SemaphoreType.DMA,
        pltpu.SemaphoreType.DMA,
        pltpu.SemaphoreType.DMA,
        pltpu.SemaphoreType.DMA,
        pltpu.SemaphoreType.DMA,
    ),
)


def _sc_agg_body(src_rr3, dst_rr3, y, z_agg, agg_out,
                 agg_acc, sblk, dblk, rows0, rows1, sg0, sg1, ss0, ss1):
    c = lax.axis_index("c")
    s = lax.axis_index("s")
    wid = c * NS + s
    pltpu.sync_copy(z_agg, agg_acc.at[pl.ds(s * ROWS_R, ROWS_R)])
    plsc.subcore_barrier()
    _gs_loop(y, agg_acc, src_rr3, dst_rr3, wid, sblk, dblk,
             rows0, rows1, sg0, sg1, ss0, ss1, NB, G_RR)
    plsc.subcore_barrier()
    pltpu.sync_copy(agg_acc.at[pl.ds(s * ROWS_R, ROWS_R)],
                    agg_out.at[c].at[pl.ds(s * ROWS_R, ROWS_R)])


_sc_agg = pl.kernel(
    _sc_agg_body,
    out_type=jax.ShapeDtypeStruct((NC, NR_P, D), f32),
    mesh=_MESH,
    scratch_types=(
        pltpu.VMEM_SHARED((NR_P, D), f32),
        pltpu.VMEM((G_RR, CW), jnp.int32),
        pltpu.VMEM((G_RR, CW), jnp.int32),
        pltpu.VMEM((CW, D), f32),
        pltpu.VMEM((CW, D), f32),
        pltpu.SemaphoreType.DMA,
        pltpu.SemaphoreType.DMA,
        pltpu.SemaphoreType.DMA,
        pltpu.SemaphoreType.DMA,
    ),
)


def _sc_sum_body(src_rs3, dst_rs3, r1, z_sum, sum2_out,
                 sum_acc, sblk, dblk, rows0, rows1, sg0, sg1, ss0, ss1):
    c = lax.axis_index("c")
    s = lax.axis_index("s")
    wid = c * NS + s
    pltpu.sync_copy(z_sum, sum_acc.at[pl.ds(s * ROWS_S, ROWS_S)])
    plsc.subcore_barrier()
    _gs_loop(r1, sum_acc, src_rs3, dst_rs3, wid, sblk, dblk,
             rows0, rows1, sg0, sg1, ss0, ss1, NB, G_RS)
    plsc.subcore_barrier()
    pltpu.sync_copy(sum_acc.at[pl.ds(s * ROWS_S, ROWS_S)],
                    sum2_out.at[c].at[pl.ds(s * ROWS_S, ROWS_S)])


_sc_sum = pl.kernel(
    _sc_sum_body,
    out_type=jax.ShapeDtypeStruct((NC, NSUB_P, D), f32),
    mesh=_MESH,
    scratch_types=(
        pltpu.VMEM_SHARED((NSUB_P, D), f32),
        pltpu.VMEM((G_RS, CW), jnp.int32),
        pltpu.VMEM((G_RS, CW), jnp.int32),
        pltpu.VMEM((CW, D), f32),
        pltpu.VMEM((CW, D), f32),
        pltpu.SemaphoreType.DMA,
        pltpu.SemaphoreType.DMA,
        pltpu.SemaphoreType.DMA,
        pltpu.SemaphoreType.DMA,
    ),
)


def _tc_a_body(deg_ref, cnt_ref, sum1_ref, xr_ref, xs_ref, wl_ref, wr_ref,
               b_ref, y1_ref, dis_ref, s1_ref, invc_ref):
    deg = deg_ref[0, :NR, 0:1] + deg_ref[1, :NR, 0:1] + 1.0
    dis = lax.rsqrt(deg)
    dis_ref[...] = dis
    y1_ref[...] = xr_ref[...] * dis
    cnt = cnt_ref[0, :NSUB, 0:1] + cnt_ref[1, :NSUB, 0:1]
    invc = 1.0 / jnp.maximum(cnt, 1.0)
    invc_ref[...] = invc
    mean1 = (sum1_ref[0, :NSUB, :] + sum1_ref[1, :NSUB, :]) * invc
    s1_ref[...] = (
        jnp.dot(mean1, wl_ref[...], preferred_element_type=f32)
        + jnp.dot(xs_ref[...], wr_ref[...], preferred_element_type=f32)
        + b_ref[...]
    )


def _tc_b_body(agg_ref, y1_ref, dis_ref, w_ref, b_ref, r1_ref, y2_ref):
    g = (agg_ref[0, :NR, :] + agg_ref[1, :NR, :] + y1_ref[...]) * dis_ref[...]
    r1 = jnp.dot(g, w_ref[...], preferred_element_type=f32) + b_ref[...]
    r1_ref[...] = r1
    y2_ref[...] = r1 * dis_ref[...]


def _tc_c_body(agg_ref, y2_ref, dis_ref, w2_ref, b2_ref,
               sum2_ref, invc_ref, s1_ref, wl2_ref, wr2_ref, bs2_ref,
               wlin_ref, blin_ref, outr_ref, outs_ref):
    g = (agg_ref[0, :NR, :] + agg_ref[1, :NR, :] + y2_ref[...]) * dis_ref[...]
    r2 = jnp.dot(g, w2_ref[...], preferred_element_type=f32) + b2_ref[...]
    outr_ref[...] = (
        jnp.dot(r2, wlin_ref[...], preferred_element_type=f32) + blin_ref[...]
    )
    mean2 = (sum2_ref[0, :NSUB, :] + sum2_ref[1, :NSUB, :]) * invc_ref[...]
    s2 = (
        jnp.dot(mean2, wl2_ref[...], preferred_element_type=f32)
        + jnp.dot(s1_ref[...], wr2_ref[...], preferred_element_type=f32)
        + bs2_ref[...]
    )
    outs_ref[...] = (
        jnp.dot(s2, wlin_ref[...], preferred_element_type=f32) + blin_ref[...]
    )


def _pad_edges(idx, total, pad_total, fill):
    pad = jnp.full((pad_total - total,), fill, jnp.int32)
    return jnp.concatenate([idx, pad])


def kernel(x_region, x_subject, edge_index_rr, edge_index_rs,
           W_gcn1, b_gcn1, W_sage_l1, W_sage_r1, b_sage1,
           W_gcn2, b_gcn2, W_sage_l2, W_sage_r2, b_sage2,
           W_lin, b_lin):
    src_rr = _pad_edges(edge_index_rr[0], ERR, ERR_P, 0)
    dst_rr = _pad_edges(edge_index_rr[1], ERR, ERR_P, NR)
    src_rs = _pad_edges(edge_index_rs[0], ERS, ERS_P, 0)
    dst_rs = _pad_edges(edge_index_rs[1], ERS, ERS_P, NSUB)
    src_rr = src_rr.reshape(NW, NCH_RR, CW)
    dst_rr = dst_rr.reshape(NW, NCH_RR, CW)
    src_rs = src_rs.reshape(NW, NCH_RS, CW)
    dst_rs = dst_rs.reshape(NW, NCH_RS, CW)

    z_deg = jnp.zeros((ROWS_R, 16), f32)
    z_cnt = jnp.zeros((ROWS_S, 16), f32)
    z_sum = jnp.zeros((ROWS_S, D), f32)
    z_agg = jnp.zeros((ROWS_R, D), f32)
    ones128 = jnp.ones((CW, 16), f32)

    deg_p, cnt_p, sum1_p = _sc_pass1(
        dst_rr, src_rs, dst_rs, x_region, z_deg, z_cnt, z_sum, ones128
    )

    y1, dis, s1, invc = pl.pallas_call(
        _tc_a_body,
        out_shape=(
            jax.ShapeDtypeStruct((NR, D), f32),
            jax.ShapeDtypeStruct((NR, 1), f32),
            jax.ShapeDtypeStruct((NSUB, H), f32),
            jax.ShapeDtypeStruct((NSUB, 1), f32),
        ),
    )(deg_p, cnt_p, sum1_p, x_region, x_subject,
      W_sage_l1, W_sage_r1, b_sage1.reshape(1, H))

    agg1_p = _sc_agg(src_rr, dst_rr, y1, z_agg)

    r1, y2 = pl.pallas_call(
        _tc_b_body,
        out_shape=(
            jax.ShapeDtypeStruct((NR, H), f32),
            jax.ShapeDtypeStruct((NR, H), f32),
        ),
    )(agg1_p, y1, dis, W_gcn1, b_gcn1.reshape(1, H))

    agg2_p = _sc_agg(src_rr, dst_rr, y2, z_agg)
    sum2_p = _sc_sum(src_rs, dst_rs, r1, z_sum)

    out_region, out_subject = pl.pallas_call(
        _tc_c_body,
        out_shape=(
            jax.ShapeDtypeStruct((NR, OUTD), f32),
            jax.ShapeDtypeStruct((NSUB, OUTD), f32),
        ),
    )(agg2_p, y2, dis, W_gcn2, b_gcn2.reshape(1, H),
      sum2_p, invc, s1, W_sage_l2, W_sage_r2, b_sage2.reshape(1, H),
      W_lin, b_lin.reshape(1, OUTD))

    return (out_region, out_subject)
